# Initial kernel scaffold; baseline (speedup 1.0000x reference)
import functools

import jax
import jax.numpy as jnp
from jax import lax
from jax.experimental import pallas as pl
from jax.experimental.pallas import tpu as pltpu
from jax.experimental.pallas import tpu_sc as plsc

N = 10000
E = 160000
D = 256
DO = 64
DH = D // 2
NC, NS, L = 2, 16, 16

BLK = 1000
CH = 80
CHD = 40
ZR = 125


def _gru_body(w_ref, wit_ref, wht_ref, bi_ref, bh_ref, o_ref):
    w = w_ref[...]
    gi = jnp.dot(w, wit_ref[...], preferred_element_type=jnp.float32) + bi_ref[...]
    gh = jnp.dot(w, wht_ref[...], preferred_element_type=jnp.float32) + bh_ref[...]
    r = jax.nn.sigmoid(gi[:, :D] + gh[:, :D])
    z = jax.nn.sigmoid(gi[:, D:2 * D] + gh[:, D:2 * D])
    n = jnp.tanh(gi[:, 2 * D:] + r * gh[:, 2 * D:])
    o_ref[...] = (1.0 - z) * n + z * w


def _gru(w, wi, wh, bi, bh):
    return pl.pallas_call(
        _gru_body,
        out_shape=jax.ShapeDtypeStruct((D, D), jnp.float32),
    )(w, wi.T, wh.T, bi.reshape(1, -1), bh.reshape(1, -1))


def _mm1_body(x_ref, w_ref, o_ref):
    o_ref[...] = jnp.dot(x_ref[...], w_ref[...], preferred_element_type=jnp.float32)


def _mm1(x, w):
    return pl.pallas_call(
        _mm1_body,
        grid=(N // BLK,),
        in_specs=[
            pl.BlockSpec((BLK, D), lambda i: (i, 0)),
            pl.BlockSpec((D, D), lambda i: (0, 0)),
        ],
        out_specs=pl.BlockSpec((BLK, D), lambda i: (i, 0)),
        out_shape=jax.ShapeDtypeStruct((N, D), jnp.float32),
    )(x, w)


def _scale1_body(d_ref, h_ref, hs_ref, dinv_ref):
    deg = 1.0 + d_ref[0] + d_ref[1]
    dinv = 1.0 / jnp.sqrt(deg)
    dinv_ref[...] = dinv
    hs = dinv[:, :1] * h_ref[...]
    hs_ref[0] = hs[:, :DH]
    hs_ref[1] = hs[:, DH:]


def _scale1(degp, h):
    return pl.pallas_call(
        _scale1_body,
        grid=(N // BLK,),
        in_specs=[
            pl.BlockSpec((NC, BLK, L), lambda i: (0, i, 0)),
            pl.BlockSpec((BLK, D), lambda i: (i, 0)),
        ],
        out_specs=[
            pl.BlockSpec((NC, BLK, DH), lambda i: (0, i, 0)),
            pl.BlockSpec((BLK, L), lambda i: (i, 0)),
        ],
        out_shape=[
            jax.ShapeDtypeStruct((NC, N, DH), jnp.float32),
            jax.ShapeDtypeStruct((N, L), jnp.float32),
        ],
    )(degp, h)


def _ep1_body(acc_ref, h_ref, dinv_ref, w_ref, b_ref, o_ref):
    dinv = dinv_ref[:, :1]
    acc = jnp.concatenate([acc_ref[0], acc_ref[1]], axis=1)
    t = dinv * acc + (dinv * dinv) * h_ref[...]
    t = jnp.maximum(t, 0.0)
    o_ref[...] = jnp.dot(t, w_ref[...], preferred_element_type=jnp.float32) + b_ref[...]


def _ep1(acc, h, dinv, w, b):
    return pl.pallas_call(
        _ep1_body,
        grid=(N // BLK,),
        in_specs=[
            pl.BlockSpec((NC, BLK, DH), lambda i: (0, i, 0)),
            pl.BlockSpec((BLK, D), lambda i: (i, 0)),
            pl.BlockSpec((BLK, L), lambda i: (i, 0)),
            pl.BlockSpec((D, D), lambda i: (0, 0)),
            pl.BlockSpec((1, D), lambda i: (0, 0)),
        ],
        out_specs=pl.BlockSpec((BLK, D), lambda i: (i, 0)),
        out_shape=jax.ShapeDtypeStruct((N, D), jnp.float32),
    )(acc, h, dinv, w.T, b.reshape(1, -1))


def _mm2_body(h_ref, w_ref, dinv_ref, g_ref, hs_ref):
    g = jnp.dot(h_ref[...], w_ref[...], preferred_element_type=jnp.float32)
    g_ref[...] = g
    hs = dinv_ref[:, :1] * g
    hs_ref[0] = hs[:, :DH]
    hs_ref[1] = hs[:, DH:]


def _mm2(h, w, dinv):
    return pl.pallas_call(
        _mm2_body,
        grid=(N // BLK,),
        in_specs=[
            pl.BlockSpec((BLK, D), lambda i: (i, 0)),
            pl.BlockSpec((D, D), lambda i: (0, 0)),
            pl.BlockSpec((BLK, L), lambda i: (i, 0)),
        ],
        out_specs=[
            pl.BlockSpec((BLK, D), lambda i: (i, 0)),
            pl.BlockSpec((NC, BLK, DH), lambda i: (0, i, 0)),
        ],
        out_shape=[
            jax.ShapeDtypeStruct((N, D), jnp.float32),
            jax.ShapeDtypeStruct((NC, N, DH), jnp.float32),
        ],
    )(h, w, dinv)


def _ep2_body(acc_ref, g_ref, dinv_ref, w_ref, b_ref, o_ref):
    dinv = dinv_ref[:, :1]
    acc = jnp.concatenate([acc_ref[0], acc_ref[1]], axis=1)
    t = dinv * acc + (dinv * dinv) * g_ref[...]
    o_ref[...] = jax.nn.sigmoid(
        jnp.dot(t, w_ref[...], preferred_element_type=jnp.float32) + b_ref[...])


def _ep2(acc, g, dinv, w, b):
    return pl.pallas_call(
        _ep2_body,
        grid=(N // BLK,),
        in_specs=[
            pl.BlockSpec((NC, BLK, DH), lambda i: (0, i, 0)),
            pl.BlockSpec((BLK, D), lambda i: (i, 0)),
            pl.BlockSpec((BLK, L), lambda i: (i, 0)),
            pl.BlockSpec((D, DO), lambda i: (0, 0)),
            pl.BlockSpec((1, DO), lambda i: (0, 0)),
        ],
        out_specs=pl.BlockSpec((BLK, DO), lambda i: (i, 0)),
        out_shape=jax.ShapeDtypeStruct((N, DO), jnp.float32),
    )(acc, g, dinv, w.T, b.reshape(1, -1))


_MESH = plsc.VectorSubcoreMesh(core_axis_name="c", subcore_axis_name="s")


@functools.partial(
    pl.kernel,
    out_type=jax.ShapeDtypeStruct((NC, N, L), jnp.float32),
    mesh=_MESH,
    scratch_types=[
        pltpu.VMEM((CHD,), jnp.int32),
        pltpu.VMEM((CHD, L), jnp.float32),
        pltpu.VMEM((ZR, L), jnp.float32),
        pltpu.VMEM_SHARED((N, L), jnp.float32),
    ],
)
def _deg_sc(dst_hbm, out_hbm, dstv, ones_v, zbuf, acc):
    c = lax.axis_index("c")
    s = lax.axis_index("s")

    @pl.loop(0, CHD)
    def _(i):
        ones_v[i, :] = jnp.full((L,), 1.0, jnp.float32)

    @pl.loop(0, ZR)
    def _(i):
        zbuf[i, :] = jnp.zeros((L,), jnp.float32)

    rbase = s * (N // NS)

    @pl.loop(0, (N // NS) // ZR)
    def _(j):
        pltpu.sync_copy(zbuf, acc.at[pl.ds(rbase + j * ZR, ZR)])

    plsc.subcore_barrier()

    ebase = (c * NS + s) * (E // (NC * NS))

    @pl.loop(0, (E // (NC * NS)) // CHD)
    def _(j):
        pltpu.sync_copy(dst_hbm.at[pl.ds(ebase + j * CHD, CHD)], dstv)
        pltpu.sync_copy(ones_v, acc.at[dstv], add=True)

    plsc.subcore_barrier()

    @pl.loop(0, (N // NS) // ZR)
    def _(j):
        r0 = rbase + j * ZR
        pltpu.sync_copy(acc.at[pl.ds(r0, ZR)], out_hbm.at[c].at[pl.ds(r0, ZR)])


@functools.partial(
    pl.kernel,
    out_type=jax.ShapeDtypeStruct((NC, N, DH), jnp.float32),
    mesh=_MESH,
    scratch_types=[
        pltpu.VMEM((CH,), jnp.int32),
        pltpu.VMEM((CH,), jnp.int32),
        pltpu.VMEM((CH, DH), jnp.float32),
        pltpu.VMEM((ZR, DH), jnp.float32),
        pltpu.VMEM_SHARED((N, DH), jnp.float32),
        pltpu.SemaphoreType.DMA,
    ],
)
def _scatter_sc(hs_hbm, src_hbm, dst_hbm, out_hbm, srcv, dstv, rows, zbuf, acc, sem):
    c = lax.axis_index("c")
    s = lax.axis_index("s")

    @pl.loop(0, ZR)
    def _(i):
        @pl.loop(0, DH // L)
        def _(k):
            zbuf[i, pl.ds(k * L, L)] = jnp.zeros((L,), jnp.float32)

    rbase = s * (N // NS)

    @pl.loop(0, (N // NS) // ZR)
    def _(j):
        pltpu.sync_copy(zbuf, acc.at[pl.ds(rbase + j * ZR, ZR)])

    plsc.subcore_barrier()

    ebase = s * (E // NS)

    @pl.loop(0, (E // NS) // CH)
    def _(j):
        e0 = ebase + j * CH
        pltpu.sync_copy(src_hbm.at[pl.ds(e0, CH)], srcv)
        pltpu.sync_copy(dst_hbm.at[pl.ds(e0, CH)], dstv)
        pltpu.async_copy(hs_hbm.at[c].at[srcv], rows, sem).wait()
        pltpu.sync_copy(rows, acc.at[dstv], add=True)

    plsc.subcore_barrier()

    @pl.loop(0, (N // NS) // ZR)
    def _(j):
        r0 = rbase + j * ZR
        pltpu.sync_copy(acc.at[pl.ds(r0, ZR)], out_hbm.at[c].at[pl.ds(r0, ZR)])


def kernel(x, edge_index, weight1, gru1_wi, gru1_wh, gru1_bi, gru1_bh,
           weight2, gru2_wi, gru2_wh, gru2_bi, gru2_bh,
           lin0_w, lin0_b, lin1_w, lin1_b):
    ei = edge_index.astype(jnp.int32)
    src = ei[0]
    dst = ei[1]

    wt1 = _gru(weight1, gru1_wi, gru1_wh, gru1_bi, gru1_bh)
    wt2 = _gru(weight2, gru2_wi, gru2_wh, gru2_bi, gru2_bh)

    degp = _deg_sc(dst)
    h1 = _mm1(x, wt1)
    hs1, dinv = _scale1(degp, h1)
    acc1 = _scatter_sc(hs1, src, dst)
    h2 = _ep1(acc1, h1, dinv, lin0_w, lin0_b)
    g2, hs2 = _mm2(h2, wt2, dinv)
    acc2 = _scatter_sc(hs2, src, dst)
    return _ep2(acc2, g2, dinv, lin1_w, lin1_b)


# speedup vs baseline: 7.7034x; 7.7034x over previous
import functools

import jax
import jax.numpy as jnp
from jax import lax
from jax.experimental import pallas as pl
from jax.experimental.pallas import tpu as pltpu
from jax.experimental.pallas import tpu_sc as plsc

N = 10000
E = 160000
D = 256
DO = 64
DH = D // 2
NC, NS, L = 2, 16, 16

BLK = 1000
CH = 80
CHD = 40
NP = 10240
ZR = 128


def _gru_body(w_ref, wit_ref, wht_ref, bi_ref, bh_ref, o_ref):
    w = w_ref[...]
    gi = jnp.dot(w, wit_ref[...], preferred_element_type=jnp.float32) + bi_ref[...]
    gh = jnp.dot(w, wht_ref[...], preferred_element_type=jnp.float32) + bh_ref[...]
    r = jax.nn.sigmoid(gi[:, :D] + gh[:, :D])
    z = jax.nn.sigmoid(gi[:, D:2 * D] + gh[:, D:2 * D])
    n = jnp.tanh(gi[:, 2 * D:] + r * gh[:, 2 * D:])
    o_ref[...] = (1.0 - z) * n + z * w


def _gru(w, wi, wh, bi, bh):
    return pl.pallas_call(
        _gru_body,
        out_shape=jax.ShapeDtypeStruct((D, D), jnp.float32),
    )(w, wi.T, wh.T, bi.reshape(1, -1), bh.reshape(1, -1))


def _mm1_body(x_ref, w_ref, o_ref):
    o_ref[...] = jnp.dot(x_ref[...], w_ref[...], preferred_element_type=jnp.float32)


def _mm1(x, w):
    return pl.pallas_call(
        _mm1_body,
        grid=(N // BLK,),
        in_specs=[
            pl.BlockSpec((BLK, D), lambda i: (i, 0)),
            pl.BlockSpec((D, D), lambda i: (0, 0)),
        ],
        out_specs=pl.BlockSpec((BLK, D), lambda i: (i, 0)),
        out_shape=jax.ShapeDtypeStruct((N, D), jnp.float32),
    )(x, w)


def _scale1_body(d_ref, h_ref, hs_ref, dinv_ref):
    deg = 1.0 + d_ref[0] + d_ref[1]
    dinv = 1.0 / jnp.sqrt(deg)
    dinv_ref[...] = dinv
    hs = dinv[:, :1] * h_ref[...]
    hs_ref[0] = hs[:, :DH]
    hs_ref[1] = hs[:, DH:]


def _scale1(degp, h):
    return pl.pallas_call(
        _scale1_body,
        grid=(N // BLK,),
        in_specs=[
            pl.BlockSpec((NC, BLK, L), lambda i: (0, i, 0)),
            pl.BlockSpec((BLK, D), lambda i: (i, 0)),
        ],
        out_specs=[
            pl.BlockSpec((NC, BLK, DH), lambda i: (0, i, 0)),
            pl.BlockSpec((BLK, L), lambda i: (i, 0)),
        ],
        out_shape=[
            jax.ShapeDtypeStruct((NC, N, DH), jnp.float32),
            jax.ShapeDtypeStruct((N, L), jnp.float32),
        ],
    )(degp, h)


def _ep1_body(acc_ref, h_ref, dinv_ref, w_ref, b_ref, o_ref):
    dinv = dinv_ref[:, :1]
    acc = jnp.concatenate([acc_ref[0], acc_ref[1]], axis=1)
    t = dinv * acc + (dinv * dinv) * h_ref[...]
    t = jnp.maximum(t, 0.0)
    o_ref[...] = jnp.dot(t, w_ref[...], preferred_element_type=jnp.float32) + b_ref[...]


def _ep1(acc, h, dinv, w, b):
    return pl.pallas_call(
        _ep1_body,
        grid=(N // BLK,),
        in_specs=[
            pl.BlockSpec((NC, BLK, DH), lambda i: (0, i, 0)),
            pl.BlockSpec((BLK, D), lambda i: (i, 0)),
            pl.BlockSpec((BLK, L), lambda i: (i, 0)),
            pl.BlockSpec((D, D), lambda i: (0, 0)),
            pl.BlockSpec((1, D), lambda i: (0, 0)),
        ],
        out_specs=pl.BlockSpec((BLK, D), lambda i: (i, 0)),
        out_shape=jax.ShapeDtypeStruct((N, D), jnp.float32),
    )(acc, h, dinv, w.T, b.reshape(1, -1))


def _mm2_body(h_ref, w_ref, dinv_ref, g_ref, hs_ref):
    g = jnp.dot(h_ref[...], w_ref[...], preferred_element_type=jnp.float32)
    g_ref[...] = g
    hs = dinv_ref[:, :1] * g
    hs_ref[0] = hs[:, :DH]
    hs_ref[1] = hs[:, DH:]


def _mm2(h, w, dinv):
    return pl.pallas_call(
        _mm2_body,
        grid=(N // BLK,),
        in_specs=[
            pl.BlockSpec((BLK, D), lambda i: (i, 0)),
            pl.BlockSpec((D, D), lambda i: (0, 0)),
            pl.BlockSpec((BLK, L), lambda i: (i, 0)),
        ],
        out_specs=[
            pl.BlockSpec((BLK, D), lambda i: (i, 0)),
            pl.BlockSpec((NC, BLK, DH), lambda i: (0, i, 0)),
        ],
        out_shape=[
            jax.ShapeDtypeStruct((N, D), jnp.float32),
            jax.ShapeDtypeStruct((NC, N, DH), jnp.float32),
        ],
    )(h, w, dinv)


def _ep2_body(acc_ref, g_ref, dinv_ref, w_ref, b_ref, o_ref):
    dinv = dinv_ref[:, :1]
    acc = jnp.concatenate([acc_ref[0], acc_ref[1]], axis=1)
    t = dinv * acc + (dinv * dinv) * g_ref[...]
    o_ref[...] = jax.nn.sigmoid(
        jnp.dot(t, w_ref[...], preferred_element_type=jnp.float32) + b_ref[...])


def _ep2(acc, g, dinv, w, b):
    return pl.pallas_call(
        _ep2_body,
        grid=(N // BLK,),
        in_specs=[
            pl.BlockSpec((NC, BLK, DH), lambda i: (0, i, 0)),
            pl.BlockSpec((BLK, D), lambda i: (i, 0)),
            pl.BlockSpec((BLK, L), lambda i: (i, 0)),
            pl.BlockSpec((D, DO), lambda i: (0, 0)),
            pl.BlockSpec((1, DO), lambda i: (0, 0)),
        ],
        out_specs=pl.BlockSpec((BLK, DO), lambda i: (i, 0)),
        out_shape=jax.ShapeDtypeStruct((N, DO), jnp.float32),
    )(acc, g, dinv, w.T, b.reshape(1, -1))


@functools.lru_cache(maxsize=None)
def _sc_mesh():
    return plsc.VectorSubcoreMesh(core_axis_name="c", subcore_axis_name="s",
                                  num_cores=NC, num_subcores=NS)


@functools.lru_cache(maxsize=None)
def _deg_sc_kernel():
    return pl.kernel(
        _deg_sc_body,
        out_type=jax.ShapeDtypeStruct((NC, NP, L), jnp.float32),
        mesh=_sc_mesh(),
        scratch_types=[
            pltpu.VMEM((CHD,), jnp.int32),
            pltpu.VMEM((CHD, L), jnp.float32),
            pltpu.VMEM((ZR, L), jnp.float32),
            pltpu.VMEM_SHARED((NP, L), jnp.float32),
        ],
    )


def _deg_sc_body(dst_hbm, out_hbm, dstv, ones_v, zbuf, acc):
    c = lax.axis_index("c")
    s = lax.axis_index("s")

    @pl.loop(0, CHD)
    def _(i):
        ones_v[i, :] = jnp.full((L,), 1.0, jnp.float32)

    @pl.loop(0, ZR)
    def _(i):
        zbuf[i, :] = jnp.zeros((L,), jnp.float32)

    rbase = s * (NP // NS)

    @pl.loop(0, (NP // NS) // ZR)
    def _(j):
        pltpu.sync_copy(zbuf, acc.at[pl.ds(rbase + j * ZR, ZR)])

    plsc.subcore_barrier()

    ebase = (c * NS + s) * (E // (NC * NS))

    @pl.loop(0, (E // (NC * NS)) // CHD)
    def _(j):
        pltpu.sync_copy(dst_hbm.at[pl.ds(ebase + j * CHD, CHD)], dstv)
        pltpu.sync_copy(ones_v, acc.at[dstv], add=True)

    plsc.subcore_barrier()

    @pl.loop(0, (NP // NS) // ZR)
    def _(j):
        r0 = rbase + j * ZR
        pltpu.sync_copy(acc.at[pl.ds(r0, ZR)], out_hbm.at[c].at[pl.ds(r0, ZR)])


@functools.lru_cache(maxsize=None)
def _scatter_sc_kernel():
    return pl.kernel(
        _scatter_sc_body,
        out_type=jax.ShapeDtypeStruct((NC, NP, DH), jnp.float32),
        mesh=_sc_mesh(),
        scratch_types=[
            pltpu.VMEM((CH,), jnp.int32),
            pltpu.VMEM((CH,), jnp.int32),
            pltpu.VMEM((CH, DH), jnp.float32),
            pltpu.VMEM((ZR, DH), jnp.float32),
            pltpu.VMEM_SHARED((NP, DH), jnp.float32),
            pltpu.SemaphoreType.DMA,
        ],
    )


def _scatter_sc_body(hs_hbm, src_hbm, dst_hbm, out_hbm, srcv, dstv, rows, zbuf, acc, sem):
    c = lax.axis_index("c")
    s = lax.axis_index("s")

    @pl.loop(0, ZR)
    def _(i):
        @pl.loop(0, DH // L)
        def _(k):
            zbuf[i, pl.ds(k * L, L)] = jnp.zeros((L,), jnp.float32)

    rbase = s * (NP // NS)

    @pl.loop(0, (NP // NS) // ZR)
    def _(j):
        pltpu.sync_copy(zbuf, acc.at[pl.ds(rbase + j * ZR, ZR)])

    plsc.subcore_barrier()

    ebase = s * (E // NS)

    @pl.loop(0, (E // NS) // CH)
    def _(j):
        e0 = ebase + j * CH
        pltpu.sync_copy(src_hbm.at[pl.ds(e0, CH)], srcv)
        pltpu.sync_copy(dst_hbm.at[pl.ds(e0, CH)], dstv)
        pltpu.async_copy(hs_hbm.at[c].at[srcv], rows, sem).wait()
        pltpu.sync_copy(rows, acc.at[dstv], add=True)

    plsc.subcore_barrier()

    @pl.loop(0, (NP // NS) // ZR)
    def _(j):
        r0 = rbase + j * ZR
        pltpu.sync_copy(acc.at[pl.ds(r0, ZR)], out_hbm.at[c].at[pl.ds(r0, ZR)])


def kernel(x, edge_index, weight1, gru1_wi, gru1_wh, gru1_bi, gru1_bh,
           weight2, gru2_wi, gru2_wh, gru2_bi, gru2_bh,
           lin0_w, lin0_b, lin1_w, lin1_b):
    ei = edge_index.astype(jnp.int32)
    src = ei[0]
    dst = ei[1]

    wt1 = _gru(weight1, gru1_wi, gru1_wh, gru1_bi, gru1_bh)
    wt2 = _gru(weight2, gru2_wi, gru2_wh, gru2_bi, gru2_bh)

    degp = _deg_sc_kernel()(dst)
    h1 = _mm1(x, wt1)
    hs1, dinv = _scale1(degp, h1)
    acc1 = _scatter_sc_kernel()(hs1, src, dst)
    h2 = _ep1(acc1, h1, dinv, lin0_w, lin0_b)
    g2, hs2 = _mm2(h2, wt2, dinv)
    acc2 = _scatter_sc_kernel()(hs2, src, dst)
    return _ep2(acc2, g2, dinv, lin1_w, lin1_b)


# speedup vs baseline: 18.1909x; 2.3614x over previous
import functools

import jax
import jax.numpy as jnp
from jax import lax
from jax.experimental import pallas as pl
from jax.experimental.pallas import tpu as pltpu
from jax.experimental.pallas import tpu_sc as plsc

N = 10000
E = 160000
D = 256
DO = 64
DH = D // 2
NC, NS, L = 2, 16, 16

BLK = 1000
CH = 125
NCHT = E // (NS * CH)
NCHD = E // (NC * NS * CH)
NCH2 = NCHT // 2
NP = 10240
RPT = NP // NS
ZR = 128
ZB = 40


def _gru_body(w_ref, wit_ref, wht_ref, bi_ref, bh_ref, o_ref):
    w = w_ref[...]
    gi = jnp.dot(w, wit_ref[...], preferred_element_type=jnp.float32) + bi_ref[...]
    gh = jnp.dot(w, wht_ref[...], preferred_element_type=jnp.float32) + bh_ref[...]
    r = jax.nn.sigmoid(gi[:, :D] + gh[:, :D])
    z = jax.nn.sigmoid(gi[:, D:2 * D] + gh[:, D:2 * D])
    n = jnp.tanh(gi[:, 2 * D:] + r * gh[:, 2 * D:])
    o_ref[...] = (1.0 - z) * n + z * w


def _gru(w, wi, wh, bi, bh):
    return pl.pallas_call(
        _gru_body,
        out_shape=jax.ShapeDtypeStruct((D, D), jnp.float32),
    )(w, wi.T, wh.T, bi.reshape(1, -1), bh.reshape(1, -1))


def _mm1_body(x_ref, w_ref, o_ref):
    o_ref[...] = jnp.dot(x_ref[...], w_ref[...], preferred_element_type=jnp.float32)


def _mm1(x, w):
    return pl.pallas_call(
        _mm1_body,
        grid=(N // BLK,),
        in_specs=[
            pl.BlockSpec((BLK, D), lambda i: (i, 0)),
            pl.BlockSpec((D, D), lambda i: (0, 0)),
        ],
        out_specs=pl.BlockSpec((BLK, D), lambda i: (i, 0)),
        out_shape=jax.ShapeDtypeStruct((N, D), jnp.float32),
    )(x, w)


def _scale1_body(d_ref, h_ref, hs_ref, dinv_ref):
    deg = 1.0 + d_ref[0] + d_ref[1]
    dinv = 1.0 / jnp.sqrt(deg)
    dinv_ref[...] = dinv
    hs = dinv[:, :1] * h_ref[...]
    hs_ref[0] = hs[:, :DH]
    hs_ref[1] = hs[:, DH:]


def _scale1(degp, h):
    return pl.pallas_call(
        _scale1_body,
        grid=(N // BLK,),
        in_specs=[
            pl.BlockSpec((NC, BLK, L), lambda i: (0, i, 0)),
            pl.BlockSpec((BLK, D), lambda i: (i, 0)),
        ],
        out_specs=[
            pl.BlockSpec((NC, BLK, DH), lambda i: (0, i, 0)),
            pl.BlockSpec((BLK, L), lambda i: (i, 0)),
        ],
        out_shape=[
            jax.ShapeDtypeStruct((NC, NP, DH), jnp.float32),
            jax.ShapeDtypeStruct((N, L), jnp.float32),
        ],
    )(degp, h)


def _ep1_body(acc_ref, h_ref, dinv_ref, w_ref, b_ref, o_ref):
    dinv = dinv_ref[:, :1]
    acc = jnp.concatenate([acc_ref[0], acc_ref[1]], axis=1)
    t = dinv * acc + (dinv * dinv) * h_ref[...]
    t = jnp.maximum(t, 0.0)
    o_ref[...] = jnp.dot(t, w_ref[...], preferred_element_type=jnp.float32) + b_ref[...]


def _ep1(acc, h, dinv, w, b):
    return pl.pallas_call(
        _ep1_body,
        grid=(N // BLK,),
        in_specs=[
            pl.BlockSpec((NC, BLK, DH), lambda i: (0, i, 0)),
            pl.BlockSpec((BLK, D), lambda i: (i, 0)),
            pl.BlockSpec((BLK, L), lambda i: (i, 0)),
            pl.BlockSpec((D, D), lambda i: (0, 0)),
            pl.BlockSpec((1, D), lambda i: (0, 0)),
        ],
        out_specs=pl.BlockSpec((BLK, D), lambda i: (i, 0)),
        out_shape=jax.ShapeDtypeStruct((N, D), jnp.float32),
    )(acc, h, dinv, w.T, b.reshape(1, -1))


def _mm2_body(h_ref, w_ref, dinv_ref, g_ref, hs_ref):
    g = jnp.dot(h_ref[...], w_ref[...], preferred_element_type=jnp.float32)
    g_ref[...] = g
    hs = dinv_ref[:, :1] * g
    hs_ref[0] = hs[:, :DH]
    hs_ref[1] = hs[:, DH:]


def _mm2(h, w, dinv):
    return pl.pallas_call(
        _mm2_body,
        grid=(N // BLK,),
        in_specs=[
            pl.BlockSpec((BLK, D), lambda i: (i, 0)),
            pl.BlockSpec((D, D), lambda i: (0, 0)),
            pl.BlockSpec((BLK, L), lambda i: (i, 0)),
        ],
        out_specs=[
            pl.BlockSpec((BLK, D), lambda i: (i, 0)),
            pl.BlockSpec((NC, BLK, DH), lambda i: (0, i, 0)),
        ],
        out_shape=[
            jax.ShapeDtypeStruct((N, D), jnp.float32),
            jax.ShapeDtypeStruct((NC, NP, DH), jnp.float32),
        ],
    )(h, w, dinv)


def _ep2_body(acc_ref, g_ref, dinv_ref, w_ref, b_ref, o_ref):
    dinv = dinv_ref[:, :1]
    acc = jnp.concatenate([acc_ref[0], acc_ref[1]], axis=1)
    t = dinv * acc + (dinv * dinv) * g_ref[...]
    o_ref[...] = jax.nn.sigmoid(
        jnp.dot(t, w_ref[...], preferred_element_type=jnp.float32) + b_ref[...])


def _ep2(acc, g, dinv, w, b):
    return pl.pallas_call(
        _ep2_body,
        grid=(N // BLK,),
        in_specs=[
            pl.BlockSpec((NC, BLK, DH), lambda i: (0, i, 0)),
            pl.BlockSpec((BLK, D), lambda i: (i, 0)),
            pl.BlockSpec((BLK, L), lambda i: (i, 0)),
            pl.BlockSpec((D, DO), lambda i: (0, 0)),
            pl.BlockSpec((1, DO), lambda i: (0, 0)),
        ],
        out_specs=pl.BlockSpec((BLK, DO), lambda i: (i, 0)),
        out_shape=jax.ShapeDtypeStruct((N, DO), jnp.float32),
    )(acc, g, dinv, w.T, b.reshape(1, -1))


@functools.lru_cache(maxsize=None)
def _sc_mesh():
    return plsc.VectorSubcoreMesh(core_axis_name="c", subcore_axis_name="s",
                                  num_cores=NC, num_subcores=NS)


@functools.lru_cache(maxsize=None)
def _deg_sc_kernel():
    return pl.kernel(
        _deg_sc_body,
        out_type=jax.ShapeDtypeStruct((NC, NP, L), jnp.float32),
        mesh=_sc_mesh(),
        scratch_types=[
            pltpu.VMEM((NCHD, CH), jnp.int32),
            pltpu.VMEM((CH, L), jnp.float32),
            pltpu.VMEM((ZR, L), jnp.float32),
            pltpu.VMEM_SHARED((NP, L), jnp.float32),
        ],
    )


def _deg_sc_body(dst_hbm, out_hbm, dsti, ones_v, zbuf, acc):
    c = lax.axis_index("c")
    s = lax.axis_index("s")

    pltpu.sync_copy(dst_hbm.at[pl.ds((c * NS + s) * NCHD, NCHD)], dsti)

    @pl.loop(0, CH)
    def _(i):
        ones_v[i, :] = jnp.full((L,), 1.0, jnp.float32)

    @pl.loop(0, ZR)
    def _(i):
        zbuf[i, :] = jnp.zeros((L,), jnp.float32)

    rbase = s * RPT

    @pl.loop(0, RPT // ZR)
    def _(j):
        pltpu.sync_copy(zbuf, acc.at[pl.ds(rbase + j * ZR, ZR)])

    plsc.subcore_barrier()

    @pl.loop(0, NCHD)
    def _(j):
        pltpu.sync_copy(ones_v, acc.at[dsti.at[j]], add=True)

    plsc.subcore_barrier()

    pltpu.sync_copy(acc.at[pl.ds(rbase, RPT)], out_hbm.at[c].at[pl.ds(rbase, RPT)])


@functools.lru_cache(maxsize=None)
def _scatter_sc_kernel():
    return pl.kernel(
        _scatter_sc_body,
        out_type=jax.ShapeDtypeStruct((NC, NP, DH), jnp.float32),
        mesh=_sc_mesh(),
        scratch_types=[
            pltpu.VMEM((NCH2, CH), jnp.int32),
            pltpu.VMEM((NCH2, CH), jnp.int32),
            pltpu.VMEM((CH, DH), jnp.float32),
            pltpu.VMEM((CH, DH), jnp.float32),
            pltpu.VMEM((ZB, DH), jnp.float32),
            pltpu.VMEM_SHARED((NP, DH), jnp.float32),
            pltpu.SemaphoreType.DMA,
            pltpu.SemaphoreType.DMA,
        ],
    )


def _scatter_sc_body(hs_hbm, src_hbm, dst_hbm, out_hbm,
                     srci, dsti, rows0, rows1, zbuf, acc, sem0, sem1):
    c = lax.axis_index("c")
    s = lax.axis_index("s")
    rbase = s * RPT

    @pl.loop(0, ZB)
    def _(i):
        @pl.loop(0, DH // L)
        def _(k):
            zbuf[i, pl.ds(k * L, L)] = jnp.zeros((L,), jnp.float32)

    @pl.loop(0, RPT // ZB)
    def _(j):
        pltpu.sync_copy(zbuf, acc.at[pl.ds(rbase + j * ZB, ZB)])

    plsc.subcore_barrier()

    hsc = hs_hbm.at[c]
    for h in range(2):
        cb = s * NCHT + h * NCH2
        pltpu.sync_copy(src_hbm.at[pl.ds(cb, NCH2)], srci)
        pltpu.sync_copy(dst_hbm.at[pl.ds(cb, NCH2)], dsti)

        pltpu.async_copy(hsc.at[srci.at[0]], rows0, sem0)
        pltpu.async_copy(hsc.at[srci.at[1]], rows1, sem1)

        @pl.loop(0, NCH2 - 2, step=2)
        def _(j):
            pltpu.make_async_copy(hsc.at[srci.at[j]], rows0, sem0).wait()
            pltpu.sync_copy(rows0, acc.at[dsti.at[j]], add=True)
            pltpu.async_copy(hsc.at[srci.at[j + 2]], rows0, sem0)
            pltpu.make_async_copy(hsc.at[srci.at[j + 1]], rows1, sem1).wait()
            pltpu.sync_copy(rows1, acc.at[dsti.at[j + 1]], add=True)
            pltpu.async_copy(hsc.at[srci.at[j + 3]], rows1, sem1)

        pltpu.make_async_copy(hsc.at[srci.at[NCH2 - 2]], rows0, sem0).wait()
        pltpu.sync_copy(rows0, acc.at[dsti.at[NCH2 - 2]], add=True)
        pltpu.make_async_copy(hsc.at[srci.at[NCH2 - 1]], rows1, sem1).wait()
        pltpu.sync_copy(rows1, acc.at[dsti.at[NCH2 - 1]], add=True)

    plsc.subcore_barrier()

    pltpu.sync_copy(acc.at[pl.ds(rbase, RPT)], out_hbm.at[c].at[pl.ds(rbase, RPT)])


def kernel(x, edge_index, weight1, gru1_wi, gru1_wh, gru1_bi, gru1_bh,
           weight2, gru2_wi, gru2_wh, gru2_bi, gru2_bh,
           lin0_w, lin0_b, lin1_w, lin1_b):
    ei = edge_index.astype(jnp.int32)
    src = ei[0].reshape(E // CH, CH)
    dst = ei[1].reshape(E // CH, CH)

    wt1 = _gru(weight1, gru1_wi, gru1_wh, gru1_bi, gru1_bh)
    wt2 = _gru(weight2, gru2_wi, gru2_wh, gru2_bi, gru2_bh)

    degp = _deg_sc_kernel()(dst)
    h1 = _mm1(x, wt1)
    hs1, dinv = _scale1(degp, h1)
    acc1 = _scatter_sc_kernel()(hs1, src, dst)
    h2 = _ep1(acc1, h1, dinv, lin0_w, lin0_b)
    g2, hs2 = _mm2(h2, wt2, dinv)
    acc2 = _scatter_sc_kernel()(hs2, src, dst)
    return _ep2(acc2, g2, dinv, lin1_w, lin1_b)
